# baseline (speedup 1.0000x reference)
import functools

import jax
import jax.numpy as jnp
from jax import lax
from jax.experimental import pallas as pl
from jax.experimental.pallas import tpu as pltpu
from jax.experimental.pallas import tpu_sc as plsc

_RANK = 8
_DIM = 64
_LANES = 16
_CHUNK = 128


def _lora_embed_body(steps, x_hbm, xe_hbm, wp_hbm, aflat_hbm, bs_hbm, out_hbm,
                     idx_v, xs_v, xe_v, aflat_v, wrows_v, obuf_v, bs_v,
                     sem_a, sem_w):
  num_cores = 2
  cid = lax.axis_index("c")
  sid = lax.axis_index("s")
  wid = sid * num_cores + cid

  pltpu.sync_copy(x_hbm.at[pl.ds(wid * steps, steps)], idx_v)
  pltpu.sync_copy(bs_hbm, bs_v)

  bsv = [[bs_v[pl.ds(r * _DIM + j * _LANES, _LANES)] for j in range(4)]
         for r in range(_RANK)]
  cidx = [jnp.full((_LANES,), l, jnp.int32) for l in range(_LANES)]
  ones = jnp.full((_LANES,), 1, jnp.int32)

  def step(t, carry):
    chunk = wid * steps + t
    pltpu.sync_copy(xe_hbm.at[chunk], xe_v)
    for m in range(_CHUNK // _LANES):
      xs_v[pl.ds(m * _LANES, _LANES)] = (
          lax.shift_right_logical(idx_v[t, pl.ds(m * _LANES, _LANES)], ones))
    cp_w = pltpu.async_copy(wp_hbm.at[xs_v], wrows_v, sem_w)
    cps = [pltpu.async_copy(aflat_hbm.at[xe_v.at[k]], aflat_v.at[k], sem_a)
           for k in range(_RANK)]
    for cp in cps:
      cp.wait()
    cp_w.wait()

    def krow(k, c):
      xv = idx_v[t, pl.ds(k * _LANES, _LANES)]
      for j in range(8):
        ap = aflat_v[k, pl.ds(16 * j, _LANES)]
        asp = [ap.at[cidx[l]].get(mode="promise_in_bounds")
               for l in range(_LANES)]
        for half in range(2):
          row = 16 * k + 2 * j + half
          xsp = xv.at[cidx[2 * j + half]].get(mode="promise_in_bounds")
          par = lax.convert_element_type(lax.bitwise_and(xsp, ones),
                                         jnp.float32)
          for jj in range(4):
            lo = wrows_v[row, pl.ds(jj * _LANES, _LANES)]
            hi = wrows_v[row, pl.ds(_DIM + jj * _LANES, _LANES)]
            acc = lo + par * (hi - lo)
            for r in range(_RANK):
              acc = acc + asp[8 * half + r] * bsv[r][jj]
            obuf_v[8 * k + j, pl.ds(half * _DIM + jj * _LANES, _LANES)] = acc
      return c

    lax.fori_loop(0, _CHUNK // 16, krow, 0)
    pltpu.sync_copy(obuf_v, out_hbm.at[pl.ds(chunk * (_CHUNK // 2),
                                             _CHUNK // 2)])
    return carry

  lax.fori_loop(0, steps, step, 0)


def kernel(x, weight, lora_A, lora_B):
  scaling = _RANK / (_RANK ** 0.5)
  n = x.shape[0] * x.shape[1]
  num_workers = 32
  rows_per_w = n // num_workers
  steps = rows_per_w // _CHUNK
  nchunks = num_workers * steps

  xf = x.reshape(n).astype(jnp.int32).reshape(nchunks, _CHUNK)
  xe = (xf[:, :, None] * _RANK
        + jnp.arange(_RANK, dtype=jnp.int32)).reshape(nchunks, _RANK, _CHUNK)
  bs = (lora_B * scaling).astype(jnp.float32).reshape(_RANK * _DIM)
  a_flat = lora_A.reshape(lora_A.shape[0] * _RANK)
  w_pairs = weight.reshape(weight.shape[0] // 2, 2 * _DIM)

  mesh = plsc.VectorSubcoreMesh(core_axis_name="c", subcore_axis_name="s")
  run = pl.kernel(
      functools.partial(_lora_embed_body, steps),
      out_type=jax.ShapeDtypeStruct((n // 2, 2 * _DIM), jnp.float32),
      mesh=mesh,
      scratch_types=[
          pltpu.VMEM((steps, _CHUNK), jnp.int32),
          pltpu.VMEM((_CHUNK,), jnp.int32),
          pltpu.VMEM((_RANK, _CHUNK), jnp.int32),
          pltpu.VMEM((_RANK, _CHUNK), jnp.float32),
          pltpu.VMEM((_CHUNK, 2 * _DIM), jnp.float32),
          pltpu.VMEM((_CHUNK // 2, 2 * _DIM), jnp.float32),
          pltpu.VMEM((_RANK * _DIM,), jnp.float32),
          pltpu.SemaphoreType.DMA,
          pltpu.SemaphoreType.DMA,
      ],
  )
  out = run(xf, xe, w_pairs, a_flat, bs)
  return out.reshape(x.shape[0], x.shape[1], _DIM)


# speedup vs baseline: 1.1244x; 1.1244x over previous
import functools

import jax
import jax.numpy as jnp
from jax import lax
from jax.experimental import pallas as pl
from jax.experimental.pallas import tpu as pltpu
from jax.experimental.pallas import tpu_sc as plsc

_RANK = 8
_DIM = 64
_LANES = 16
_CHUNK = 128


def _lora_embed_body(steps, x_hbm, wp_hbm, aflat_hbm, bs_hbm, out_hbm,
                     idx_v, xs0, xs1, xe0, xe1, af0, af1, wr0, wr1, ob0, ob1,
                     bs_v, sg0, sg1, so0, so1):
  num_cores = 2
  cid = lax.axis_index("c")
  sid = lax.axis_index("s")
  wid = sid * num_cores + cid

  xs = [xs0, xs1]
  xe = [xe0, xe1]
  af = [af0, af1]
  wr = [wr0, wr1]
  ob = [ob0, ob1]
  sg = [sg0, sg1]
  so = [so0, so1]

  pltpu.sync_copy(x_hbm.at[pl.ds(wid * steps, steps)], idx_v)
  pltpu.sync_copy(bs_hbm, bs_v)

  bsv = [[bs_v[pl.ds(r * _DIM + j * _LANES, _LANES)] for j in range(4)]
         for r in range(_RANK)]
  cidx = [jnp.full((_LANES,), l, jnp.int32) for l in range(_LANES)]
  ones = jnp.full((_LANES,), 1, jnp.int32)
  three = jnp.full((_LANES,), 3, jnp.int32)
  il = lax.iota(jnp.int32, _LANES)
  low8 = lax.bitwise_and(il, jnp.full((_LANES,), 7, jnp.int32))
  ihi = lax.shift_right_logical(il, three)
  patt = [jnp.full((_LANES,), 2 * m, jnp.int32) + ihi for m in range(8)]

  def build_and_fire(t, b):
    for k in range(8):
      xv = idx_v[t, pl.ds(k * _LANES, _LANES)]
      xs[b][pl.ds(k * _LANES, _LANES)] = lax.shift_right_logical(xv, ones)
      for m in range(8):
        xg = xv.at[patt[m]].get(mode="promise_in_bounds")
        xe[b][k, pl.ds(m * _LANES, _LANES)] = (
            lax.shift_left(xg, three) + low8)
    pltpu.async_copy(wp_hbm.at[xs[b]], wr[b], sg[b])
    for k in range(_RANK):
      pltpu.async_copy(aflat_hbm.at[xe[b].at[k]], af[b].at[k], sg[b])

  def wait_gathers(b):
    pltpu.make_async_copy(wp_hbm.at[xs[b]], wr[b], sg[b]).wait()
    for k in range(_RANK):
      pltpu.make_async_copy(aflat_hbm.at[xe[b].at[k]], af[b].at[k],
                            sg[b]).wait()

  def out_slice(t):
    return out_hbm.at[pl.ds((wid * steps + t) * (_CHUNK // 2), _CHUNK // 2)]

  def compute(t, b):
    def krow(k, c):
      xv = idx_v[t, pl.ds(k * _LANES, _LANES)]
      for j in range(8):
        ap = af[b][k, pl.ds(16 * j, _LANES)]
        asp = [ap.at[cidx[l]].get(mode="promise_in_bounds")
               for l in range(_LANES)]
        for half in range(2):
          row = 16 * k + 2 * j + half
          xsp = xv.at[cidx[2 * j + half]].get(mode="promise_in_bounds")
          par = lax.convert_element_type(lax.bitwise_and(xsp, ones),
                                         jnp.float32)
          for jj in range(4):
            lo = wr[b][row, pl.ds(jj * _LANES, _LANES)]
            hi = wr[b][row, pl.ds(_DIM + jj * _LANES, _LANES)]
            acc = lo + par * (hi - lo)
            for r in range(_RANK):
              acc = acc + asp[8 * half + r] * bsv[r][jj]
            ob[b][8 * k + j, pl.ds(half * _DIM + jj * _LANES, _LANES)] = acc
      return c

    lax.fori_loop(0, _CHUNK // 16, krow, 0)

  build_and_fire(0, 0)

  def body(u, c):
    t0 = 2 * u
    t1 = 2 * u + 1
    build_and_fire(t1, 1)
    wait_gathers(0)

    @pl.when(u > 0)
    def _():
      pltpu.make_async_copy(ob[0], out_slice(t0 - 2), so[0]).wait()

    compute(t0, 0)
    pltpu.async_copy(ob[0], out_slice(t0), so[0])

    @pl.when(t0 + 2 < steps)
    def _():
      build_and_fire(t0 + 2, 0)

    wait_gathers(1)

    @pl.when(u > 0)
    def _():
      pltpu.make_async_copy(ob[1], out_slice(t1 - 2), so[1]).wait()

    compute(t1, 1)
    pltpu.async_copy(ob[1], out_slice(t1), so[1])
    return c

  lax.fori_loop(0, steps // 2, body, 0)
  pltpu.make_async_copy(ob[0], out_slice(steps - 2), so[0]).wait()
  pltpu.make_async_copy(ob[1], out_slice(steps - 1), so[1]).wait()


def kernel(x, weight, lora_A, lora_B):
  scaling = _RANK / (_RANK ** 0.5)
  n = x.shape[0] * x.shape[1]
  num_workers = 32
  rows_per_w = n // num_workers
  steps = rows_per_w // _CHUNK
  nchunks = num_workers * steps

  xf = x.reshape(n).astype(jnp.int32).reshape(nchunks, _CHUNK)
  bs = (lora_B * scaling).astype(jnp.float32).reshape(_RANK * _DIM)
  a_flat = lora_A.reshape(lora_A.shape[0] * _RANK)
  w_pairs = weight.reshape(weight.shape[0] // 2, 2 * _DIM)

  mesh = plsc.VectorSubcoreMesh(core_axis_name="c", subcore_axis_name="s")
  run = pl.kernel(
      functools.partial(_lora_embed_body, steps),
      out_type=jax.ShapeDtypeStruct((n // 2, 2 * _DIM), jnp.float32),
      mesh=mesh,
      scratch_types=[
          pltpu.VMEM((steps, _CHUNK), jnp.int32),
          pltpu.VMEM((_CHUNK,), jnp.int32),
          pltpu.VMEM((_CHUNK,), jnp.int32),
          pltpu.VMEM((_RANK, _CHUNK), jnp.int32),
          pltpu.VMEM((_RANK, _CHUNK), jnp.int32),
          pltpu.VMEM((_RANK, _CHUNK), jnp.float32),
          pltpu.VMEM((_RANK, _CHUNK), jnp.float32),
          pltpu.VMEM((_CHUNK, 2 * _DIM), jnp.float32),
          pltpu.VMEM((_CHUNK, 2 * _DIM), jnp.float32),
          pltpu.VMEM((_CHUNK // 2, 2 * _DIM), jnp.float32),
          pltpu.VMEM((_CHUNK // 2, 2 * _DIM), jnp.float32),
          pltpu.VMEM((_RANK * _DIM,), jnp.float32),
          pltpu.SemaphoreType.DMA,
          pltpu.SemaphoreType.DMA,
          pltpu.SemaphoreType.DMA,
          pltpu.SemaphoreType.DMA,
      ],
  )
  out = run(xf, w_pairs, a_flat, bs)
  return out.reshape(x.shape[0], x.shape[1], _DIM)


# speedup vs baseline: 1.5532x; 1.3813x over previous
import functools

import jax
import jax.numpy as jnp
from jax import lax
from jax.experimental import pallas as pl
from jax.experimental.pallas import tpu as pltpu
from jax.experimental.pallas import tpu_sc as plsc

_RANK = 8
_DIM = 64
_LANES = 16
_CHUNK = 128


def _lora_embed_body(steps, x_hbm, w_hbm, aflat_hbm, bs_hbm, out_hbm,
                     idx_v, xe0, xe1, af0, af1, wr0, wr1, ob0, ob1,
                     bs_v, sg0, sg1, so0, so1):
  num_cores = 2
  cid = lax.axis_index("c")
  sid = lax.axis_index("s")
  wid = sid * num_cores + cid

  xe = [xe0, xe1]
  af = [af0, af1]
  wr = [wr0, wr1]
  ob = [ob0, ob1]
  sg = [sg0, sg1]
  so = [so0, so1]

  pltpu.sync_copy(x_hbm.at[pl.ds(wid * steps, steps)], idx_v)
  pltpu.sync_copy(bs_hbm, bs_v)

  bsv = [[bs_v[pl.ds(r * _DIM + j * _LANES, _LANES)] for j in range(4)]
         for r in range(_RANK)]
  cidx = [jnp.full((_LANES,), l, jnp.int32) for l in range(_LANES)]
  three = jnp.full((_LANES,), 3, jnp.int32)
  il = lax.iota(jnp.int32, _LANES)
  low8 = lax.bitwise_and(il, jnp.full((_LANES,), 7, jnp.int32))
  ihi = lax.shift_right_logical(il, three)
  patt = [jnp.full((_LANES,), 2 * m, jnp.int32) + ihi for m in range(8)]

  def build_and_fire(t, b):
    for k in range(8):
      xv = idx_v[t, pl.ds(k * _LANES, _LANES)]
      for m in range(8):
        xg = xv.at[patt[m]].get(mode="promise_in_bounds")
        xe[b][k, pl.ds(m * _LANES, _LANES)] = (
            lax.shift_left(xg, three) + low8)
    pltpu.async_copy(w_hbm.at[idx_v.at[t]], wr[b], sg[b])
    for k in range(_RANK):
      pltpu.async_copy(aflat_hbm.at[xe[b].at[k]], af[b].at[k], sg[b])

  def wait_gathers(t, b):
    pltpu.make_async_copy(w_hbm.at[idx_v.at[t]], wr[b], sg[b]).wait()
    for k in range(_RANK):
      pltpu.make_async_copy(aflat_hbm.at[xe[b].at[k]], af[b].at[k],
                            sg[b]).wait()

  def out_slice(t):
    return out_hbm.at[pl.ds((wid * steps + t) * _CHUNK, _CHUNK)]

  def compute(t, b):
    def krow(k, c):
      for j in range(8):
        ap = af[b][k, pl.ds(16 * j, _LANES)]
        asp = [ap.at[cidx[l]].get(mode="promise_in_bounds")
               for l in range(_LANES)]
        for half in range(2):
          row = 16 * k + 2 * j + half
          for jj in range(4):
            acc = wr[b][row, pl.ds(jj * _LANES, _LANES)]
            for r in range(_RANK):
              acc = acc + asp[8 * half + r] * bsv[r][jj]
            ob[b][row, pl.ds(jj * _LANES, _LANES)] = acc
      return c

    lax.fori_loop(0, _CHUNK // 16, krow, 0)

  build_and_fire(0, 0)

  def body(u, c):
    t0 = 2 * u
    t1 = 2 * u + 1
    build_and_fire(t1, 1)
    wait_gathers(t0, 0)

    @pl.when(u > 0)
    def _():
      pltpu.make_async_copy(ob[0], out_slice(t0 - 2), so[0]).wait()

    compute(t0, 0)
    pltpu.async_copy(ob[0], out_slice(t0), so[0])

    @pl.when(t0 + 2 < steps)
    def _():
      build_and_fire(t0 + 2, 0)

    wait_gathers(t1, 1)

    @pl.when(u > 0)
    def _():
      pltpu.make_async_copy(ob[1], out_slice(t1 - 2), so[1]).wait()

    compute(t1, 1)
    pltpu.async_copy(ob[1], out_slice(t1), so[1])
    return c

  lax.fori_loop(0, steps // 2, body, 0)
  pltpu.make_async_copy(ob[0], out_slice(steps - 2), so[0]).wait()
  pltpu.make_async_copy(ob[1], out_slice(steps - 1), so[1]).wait()


def kernel(x, weight, lora_A, lora_B):
  scaling = _RANK / (_RANK ** 0.5)
  n = x.shape[0] * x.shape[1]
  num_workers = 32
  rows_per_w = n // num_workers
  steps = rows_per_w // _CHUNK
  nchunks = num_workers * steps

  xf = x.reshape(n).astype(jnp.int32).reshape(nchunks, _CHUNK)
  bs = (lora_B * scaling).astype(jnp.float32).reshape(_RANK * _DIM)
  a_flat = lora_A.reshape(lora_A.shape[0] * _RANK)

  mesh = plsc.VectorSubcoreMesh(core_axis_name="c", subcore_axis_name="s")
  run = pl.kernel(
      functools.partial(_lora_embed_body, steps),
      out_type=jax.ShapeDtypeStruct((n, _DIM), jnp.float32),
      mesh=mesh,
      compiler_params=pltpu.CompilerParams(use_tc_tiling_on_sc=False),
      scratch_types=[
          pltpu.VMEM((steps, _CHUNK), jnp.int32),
          pltpu.VMEM((_RANK, _CHUNK), jnp.int32),
          pltpu.VMEM((_RANK, _CHUNK), jnp.int32),
          pltpu.VMEM((_RANK, _CHUNK), jnp.float32),
          pltpu.VMEM((_RANK, _CHUNK), jnp.float32),
          pltpu.VMEM((_CHUNK, _DIM), jnp.float32),
          pltpu.VMEM((_CHUNK, _DIM), jnp.float32),
          pltpu.VMEM((_CHUNK, _DIM), jnp.float32),
          pltpu.VMEM((_CHUNK, _DIM), jnp.float32),
          pltpu.VMEM((_RANK * _DIM,), jnp.float32),
          pltpu.SemaphoreType.DMA,
          pltpu.SemaphoreType.DMA,
          pltpu.SemaphoreType.DMA,
          pltpu.SemaphoreType.DMA,
      ],
  )
  out = run(xf, weight, a_flat, bs)
  return out.reshape(x.shape[0], x.shape[1], _DIM)
